# Initial kernel scaffold; baseline (speedup 1.0000x reference)
import jax
import jax.numpy as jnp
from jax.experimental import pallas as pl


def kernel(node_feats, edge_feats, W_proj, b_proj, W1, b1, W2, b2, bias, edge_index):
    raise NotImplementedError("write your pallas kernel here")


# speedup vs baseline: 1.9193x; 1.9193x over previous
import functools

import jax
import jax.numpy as jnp
from jax import lax
from jax.experimental import pallas as pl
from jax.experimental.pallas import tpu as pltpu
from jax.experimental.pallas import tpu_sc as plsc

_V = 10000
_E = 160000
_NODE_IN = 128
_EDGE_IN = 16
_D = 32
_EH = 128
_STEPS = 6

_NC = 2
_NS = 16
_NW = _NC * _NS
_CH = 128
_CHUNKS = 40
_EPW = _CH * _CHUNKS
_E_PAD = _NW * _EPW
_V_PAD = 10240
_RPW = _V_PAD // _NS


def _ew_body(ef_ref, w1_ref, b1_ref, w2_ref, b2_ref, out_ref):
    a = jnp.maximum(
        jnp.dot(ef_ref[...], w1_ref[...], preferred_element_type=jnp.float32)
        + b1_ref[...], 0.0)
    out_ref[...] = (
        jnp.dot(a, w2_ref[...], preferred_element_type=jnp.float32)
        + b2_ref[...])


def _proj_body(x_ref, w_ref, b_ref, out_ref):
    out_ref[...] = jnp.maximum(
        jnp.dot(x_ref[...], w_ref[...], preferred_element_type=jnp.float32)
        + b_ref[...], 0.0)


def _msg_body(hs_ref, ew_ref, t_ref, r_ref, m_ref):
    hst = jnp.dot(hs_ref[...], t_ref[...], preferred_element_type=jnp.float32)
    p = hst * ew_ref[...]
    m_ref[...] = jnp.dot(p, r_ref[...], preferred_element_type=jnp.float32)


def _combine_body(p_ref, b_ref, out_ref):
    out_ref[...] = jnp.maximum(p_ref[0] + p_ref[1] + b_ref[...], 0.0)


_MESH = plsc.VectorSubcoreMesh(core_axis_name="c", subcore_axis_name="s")
_SC_PARAMS = pltpu.CompilerParams(use_tc_tiling_on_sc=False)


@functools.partial(
    pl.kernel,
    mesh=_MESH,
    compiler_params=_SC_PARAMS,
    out_type=jax.ShapeDtypeStruct((_E_PAD, _D), jnp.float32),
    scratch_types=[
        pltpu.VMEM((_CHUNKS, _CH), jnp.int32),
        pltpu.VMEM((_CH, _D), jnp.float32),
        pltpu.SemaphoreType.DMA,
    ],
)
def _gather(h_hbm, idx_hbm, hs_hbm, idx_v, rows_v, sem):
    wid = lax.axis_index("s") * _NC + lax.axis_index("c")
    pltpu.sync_copy(idx_hbm.at[wid], idx_v)
    base = wid * _EPW

    def body(j, carry):
        pltpu.async_copy(h_hbm.at[idx_v.at[j]], rows_v, sem).wait()
        pltpu.sync_copy(rows_v, hs_hbm.at[pl.ds(base + j * _CH, _CH)])
        return carry

    lax.fori_loop(0, _CHUNKS, body, 0)


@functools.partial(
    pl.kernel,
    mesh=_MESH,
    compiler_params=_SC_PARAMS,
    out_type=jax.ShapeDtypeStruct((_NC, _V_PAD, _D), jnp.float32),
    scratch_types=[
        pltpu.VMEM((_CHUNKS, _CH), jnp.int32),
        pltpu.VMEM((_CH, _D), jnp.float32),
        pltpu.VMEM((_RPW, _D), jnp.float32),
        pltpu.VMEM_SHARED((_V_PAD, _D), jnp.float32),
    ],
)
def _scatter(m_hbm, idx_hbm, zeros_hbm, out_hbm, idx_v, rows_v, tbuf_v, acc_sh):
    cid = lax.axis_index("c")
    sid = lax.axis_index("s")
    wid = sid * _NC + cid
    pltpu.sync_copy(zeros_hbm.at[pl.ds(sid * _RPW, _RPW)], tbuf_v)
    pltpu.sync_copy(tbuf_v, acc_sh.at[pl.ds(sid * _RPW, _RPW)])
    plsc.subcore_barrier()

    pltpu.sync_copy(idx_hbm.at[wid], idx_v)
    base = wid * _EPW

    def body(j, carry):
        pltpu.sync_copy(m_hbm.at[pl.ds(base + j * _CH, _CH)], rows_v)
        pltpu.sync_copy(rows_v, acc_sh.at[idx_v.at[j]], add=True)
        return carry

    lax.fori_loop(0, _CHUNKS, body, 0)
    plsc.subcore_barrier()
    pltpu.sync_copy(acc_sh.at[pl.ds(sid * _RPW, _RPW)], tbuf_v)
    pltpu.sync_copy(tbuf_v, out_hbm.at[cid].at[pl.ds(sid * _RPW, _RPW)])


def kernel(node_feats, edge_feats, W_proj, b_proj, W1, b1, W2, b2, bias,
           edge_index):
    f32 = jnp.float32
    src = edge_index[0]
    dst = edge_index[1]

    cols = jnp.arange(_D * _D, dtype=jnp.int32)
    perm = (cols % _D) * _D + cols // _D
    W2p = W2[:, perm]
    b2p = b2[perm]

    ef_pad = jnp.zeros((_E_PAD, _EDGE_IN + 112), dtype=f32)
    ef_pad = ef_pad.at[:_E, :_EDGE_IN].set(edge_feats)
    W1p = jnp.zeros((_EDGE_IN + 112, _EH), dtype=f32).at[:_EDGE_IN].set(W1)

    x_pad = jnp.zeros((_V_PAD, _NODE_IN), dtype=f32).at[:_V].set(node_feats)

    src_rs = jnp.concatenate(
        [src, jnp.zeros((_E_PAD - _E,), jnp.int32)]).reshape(_NW, _CHUNKS, _CH)
    dst_rs = jnp.concatenate(
        [dst, jnp.full((_E_PAD - _E,), _V, jnp.int32)]).reshape(
            _NW, _CHUNKS, _CH)

    t_mat = jnp.tile(jnp.eye(_D, dtype=f32), (1, _D))
    r_mat = jnp.kron(jnp.eye(_D, dtype=f32),
                     jnp.ones((_D, 1), dtype=f32))
    bias_t = jnp.tile(bias, 128 // _D).reshape(1, 128)
    zeros_acc = jnp.zeros((_V_PAD, _D), dtype=f32)

    be = 2048
    ew3 = pl.pallas_call(
        _ew_body,
        grid=(_E_PAD // be,),
        in_specs=[
            pl.BlockSpec((be, 128), lambda i: (i, 0)),
            pl.BlockSpec((128, _EH), lambda i: (0, 0)),
            pl.BlockSpec((1, _EH), lambda i: (0, 0)),
            pl.BlockSpec((_EH, _D * _D), lambda i: (0, 0)),
            pl.BlockSpec((1, _D * _D), lambda i: (0, 0)),
        ],
        out_specs=pl.BlockSpec((be, _D * _D), lambda i: (i, 0)),
        out_shape=jax.ShapeDtypeStruct((_E_PAD, _D * _D), f32),
    )(ef_pad, W1p, b1.reshape(1, _EH), W2p, b2p.reshape(1, _D * _D))

    h = pl.pallas_call(
        _proj_body,
        in_specs=[
            pl.BlockSpec((_V_PAD, _NODE_IN), lambda: (0, 0)),
            pl.BlockSpec((_NODE_IN, _D), lambda: (0, 0)),
            pl.BlockSpec((1, _D), lambda: (0, 0)),
        ],
        out_specs=pl.BlockSpec((_V_PAD, _D), lambda: (0, 0)),
        out_shape=jax.ShapeDtypeStruct((_V_PAD, _D), f32),
    )(x_pad, W_proj, b_proj.reshape(1, _D))

    bm = 1024
    msg = pl.pallas_call(
        _msg_body,
        grid=(_E_PAD // bm,),
        in_specs=[
            pl.BlockSpec((bm, _D), lambda i: (i, 0)),
            pl.BlockSpec((bm, _D * _D), lambda i: (i, 0)),
            pl.BlockSpec((_D, _D * _D), lambda i: (0, 0)),
            pl.BlockSpec((_D * _D, _D), lambda i: (0, 0)),
        ],
        out_specs=pl.BlockSpec((bm, _D), lambda i: (i, 0)),
        out_shape=jax.ShapeDtypeStruct((_E_PAD, _D), f32),
    )

    combine = pl.pallas_call(
        _combine_body,
        in_specs=[
            pl.BlockSpec((_NC, _V_PAD * _D // 128, 128), lambda: (0, 0, 0)),
            pl.BlockSpec((1, 128), lambda: (0, 0)),
        ],
        out_specs=pl.BlockSpec((_V_PAD * _D // 128, 128), lambda: (0, 0)),
        out_shape=jax.ShapeDtypeStruct((_V_PAD * _D // 128, 128), f32),
    )

    for _ in range(_STEPS):
        hs = _gather(h, src_rs)
        m = msg(hs, ew3, t_mat, r_mat)
        p = _scatter(m, dst_rs, zeros_acc)
        h = combine(p.reshape(_NC, _V_PAD * _D // 128, 128),
                    bias_t).reshape(_V_PAD, _D)
    return h[:_V]


# speedup vs baseline: 2.3090x; 1.2030x over previous
import functools

import jax
import jax.numpy as jnp
from jax import lax
from jax.experimental import pallas as pl
from jax.experimental.pallas import tpu as pltpu
from jax.experimental.pallas import tpu_sc as plsc

_V = 10000
_E = 160000
_NODE_IN = 128
_EDGE_IN = 16
_D = 32
_EH = 128
_STEPS = 6

_NC = 2
_NS = 16
_NW = _NC * _NS
_CH = 128
_CHUNKS = 40
_EPW = _CH * _CHUNKS
_E_PAD = _NW * _EPW
_V_PAD = 10240
_RPW = _V_PAD // _NS
_WCH = 8
_WAVES = _CHUNKS // _WCH
_WROWS = _WCH * _CH


def _ew_body(ef_ref, w1_ref, b1_ref, w2_ref, b2_ref, out_ref):
    a = jnp.maximum(
        jnp.dot(ef_ref[...], w1_ref[...], preferred_element_type=jnp.float32)
        + b1_ref[...], 0.0)
    out_ref[...] = (
        jnp.dot(a, w2_ref[...], preferred_element_type=jnp.float32)
        + b2_ref[...]).astype(jnp.bfloat16)


def _proj_body(x_ref, w_ref, b_ref, out_ref):
    out_ref[...] = jnp.maximum(
        jnp.dot(x_ref[...], w_ref[...], preferred_element_type=jnp.float32)
        + b_ref[...], 0.0)


def _msg_body(hs_ref, ew_ref, t_ref, r_ref, m_ref):
    hst = jnp.dot(hs_ref[...], t_ref[...], preferred_element_type=jnp.float32)
    p = hst * ew_ref[...].astype(jnp.float32)
    m_ref[...] = jnp.dot(p, r_ref[...], preferred_element_type=jnp.float32)


def _combine_body(p_ref, b_ref, out_ref):
    out_ref[...] = jnp.maximum(p_ref[0] + p_ref[1] + b_ref[...], 0.0)


_MESH = plsc.VectorSubcoreMesh(core_axis_name="c", subcore_axis_name="s")
_SC_PARAMS = pltpu.CompilerParams(use_tc_tiling_on_sc=False)


@functools.partial(
    pl.kernel,
    mesh=_MESH,
    compiler_params=_SC_PARAMS,
    out_type=jax.ShapeDtypeStruct((_E_PAD, _D), jnp.float32),
    scratch_types=[
        pltpu.VMEM((_CHUNKS, _CH), jnp.int32),
        pltpu.VMEM((_WROWS, _D), jnp.float32),
        pltpu.VMEM((_WROWS, _D), jnp.float32),
        pltpu.SemaphoreType.DMA,
        pltpu.SemaphoreType.DMA,
    ],
)
def _gather(h_hbm, idx_hbm, hs_hbm, idx_v, buf0, buf1, gsem, wsem):
    wid = lax.axis_index("s") * _NC + lax.axis_index("c")
    pltpu.sync_copy(idx_hbm.at[wid], idx_v)
    base = wid * _EPW
    bufs = (buf0, buf1)
    wdesc = [None, None]
    for w in range(_WAVES):
        buf = bufs[w % 2]
        if wdesc[w % 2] is not None:
            wdesc[w % 2].wait()
        descs = []
        for j in range(_WCH):
            c = w * _WCH + j
            descs.append(pltpu.async_copy(
                h_hbm.at[idx_v.at[c]], buf.at[pl.ds(j * _CH, _CH)], gsem))
        for dsc in descs:
            dsc.wait()
        wdesc[w % 2] = pltpu.async_copy(
            buf, hs_hbm.at[pl.ds(base + w * _WROWS, _WROWS)], wsem)
    for d in wdesc:
        if d is not None:
            d.wait()


@functools.partial(
    pl.kernel,
    mesh=_MESH,
    compiler_params=_SC_PARAMS,
    out_type=jax.ShapeDtypeStruct((_NC, _V_PAD, _D), jnp.float32),
    scratch_types=[
        pltpu.VMEM((_CHUNKS, _CH), jnp.int32),
        pltpu.VMEM((_WROWS, _D), jnp.float32),
        pltpu.VMEM((_WROWS, _D), jnp.float32),
        pltpu.VMEM((_RPW, _D), jnp.float32),
        pltpu.VMEM_SHARED((_V_PAD, _D), jnp.float32),
        pltpu.SemaphoreType.DMA,
        pltpu.SemaphoreType.DMA,
    ],
)
def _scatter(m_hbm, idx_hbm, zeros_hbm, out_hbm, idx_v, buf0, buf1, tbuf_v,
             acc_sh, lsem, ssem):
    cid = lax.axis_index("c")
    sid = lax.axis_index("s")
    wid = sid * _NC + cid
    base = wid * _EPW
    bufs = (buf0, buf1)
    pltpu.sync_copy(idx_hbm.at[wid], idx_v)
    pltpu.sync_copy(zeros_hbm.at[pl.ds(sid * _RPW, _RPW)], tbuf_v)
    pltpu.sync_copy(tbuf_v, acc_sh.at[pl.ds(sid * _RPW, _RPW)])

    ld = [None, None]
    ld[0] = pltpu.async_copy(m_hbm.at[pl.ds(base, _WROWS)], buf0, lsem)
    plsc.subcore_barrier()
    adds = [[], []]
    for w in range(_WAVES):
        p = w % 2
        if w + 1 < _WAVES:
            pn = (w + 1) % 2
            for d in adds[pn]:
                d.wait()
            adds[pn] = []
            ld[pn] = pltpu.async_copy(
                m_hbm.at[pl.ds(base + (w + 1) * _WROWS, _WROWS)],
                bufs[pn], lsem)
        ld[p].wait()
        for j in range(_WCH):
            adds[p].append(pltpu.async_copy(
                bufs[p].at[pl.ds(j * _CH, _CH)],
                acc_sh.at[idx_v.at[w * _WCH + j]], ssem, add=True))
    for par in adds:
        for d in par:
            d.wait()
    plsc.subcore_barrier()
    pltpu.sync_copy(acc_sh.at[pl.ds(sid * _RPW, _RPW)], tbuf_v)
    pltpu.sync_copy(tbuf_v, out_hbm.at[cid].at[pl.ds(sid * _RPW, _RPW)])


def kernel(node_feats, edge_feats, W_proj, b_proj, W1, b1, W2, b2, bias,
           edge_index):
    f32 = jnp.float32
    src = edge_index[0]
    dst = edge_index[1]

    cols = jnp.arange(_D * _D, dtype=jnp.int32)
    perm = (cols % _D) * _D + cols // _D
    W2p = W2[:, perm]
    b2p = b2[perm]

    ef_pad = jnp.zeros((_E_PAD, _EDGE_IN + 112), dtype=f32)
    ef_pad = ef_pad.at[:_E, :_EDGE_IN].set(edge_feats)
    W1p = jnp.zeros((_EDGE_IN + 112, _EH), dtype=f32).at[:_EDGE_IN].set(W1)

    x_pad = jnp.zeros((_V_PAD, _NODE_IN), dtype=f32).at[:_V].set(node_feats)

    src_rs = jnp.concatenate(
        [src, jnp.zeros((_E_PAD - _E,), jnp.int32)]).reshape(_NW, _CHUNKS, _CH)
    dst_rs = jnp.concatenate(
        [dst, jnp.full((_E_PAD - _E,), _V, jnp.int32)]).reshape(
            _NW, _CHUNKS, _CH)

    t_mat = jnp.tile(jnp.eye(_D, dtype=f32), (1, _D))
    r_mat = jnp.kron(jnp.eye(_D, dtype=f32),
                     jnp.ones((_D, 1), dtype=f32))
    bias_t = jnp.tile(bias, 128 // _D).reshape(1, 128)
    zeros_acc = jnp.zeros((_V_PAD, _D), dtype=f32)

    be = 2048
    ew3 = pl.pallas_call(
        _ew_body,
        grid=(_E_PAD // be,),
        in_specs=[
            pl.BlockSpec((be, 128), lambda i: (i, 0)),
            pl.BlockSpec((128, _EH), lambda i: (0, 0)),
            pl.BlockSpec((1, _EH), lambda i: (0, 0)),
            pl.BlockSpec((_EH, _D * _D), lambda i: (0, 0)),
            pl.BlockSpec((1, _D * _D), lambda i: (0, 0)),
        ],
        out_specs=pl.BlockSpec((be, _D * _D), lambda i: (i, 0)),
        out_shape=jax.ShapeDtypeStruct((_E_PAD, _D * _D), jnp.bfloat16),
    )(ef_pad, W1p, b1.reshape(1, _EH), W2p, b2p.reshape(1, _D * _D))

    h = pl.pallas_call(
        _proj_body,
        in_specs=[
            pl.BlockSpec((_V_PAD, _NODE_IN), lambda: (0, 0)),
            pl.BlockSpec((_NODE_IN, _D), lambda: (0, 0)),
            pl.BlockSpec((1, _D), lambda: (0, 0)),
        ],
        out_specs=pl.BlockSpec((_V_PAD, _D), lambda: (0, 0)),
        out_shape=jax.ShapeDtypeStruct((_V_PAD, _D), f32),
    )(x_pad, W_proj, b_proj.reshape(1, _D))

    bm = 1024
    msg = pl.pallas_call(
        _msg_body,
        grid=(_E_PAD // bm,),
        in_specs=[
            pl.BlockSpec((bm, _D), lambda i: (i, 0)),
            pl.BlockSpec((bm, _D * _D), lambda i: (i, 0)),
            pl.BlockSpec((_D, _D * _D), lambda i: (0, 0)),
            pl.BlockSpec((_D * _D, _D), lambda i: (0, 0)),
        ],
        out_specs=pl.BlockSpec((bm, _D), lambda i: (i, 0)),
        out_shape=jax.ShapeDtypeStruct((_E_PAD, _D), f32),
    )

    combine = pl.pallas_call(
        _combine_body,
        in_specs=[
            pl.BlockSpec((_NC, _V_PAD * _D // 128, 128), lambda: (0, 0, 0)),
            pl.BlockSpec((1, 128), lambda: (0, 0)),
        ],
        out_specs=pl.BlockSpec((_V_PAD * _D // 128, 128), lambda: (0, 0)),
        out_shape=jax.ShapeDtypeStruct((_V_PAD * _D // 128, 128), f32),
    )

    for _ in range(_STEPS):
        hs = _gather(h, src_rs)
        m = msg(hs, ew3, t_mat, r_mat)
        p = _scatter(m, dst_rs, zeros_acc)
        h = combine(p.reshape(_NC, _V_PAD * _D // 128, 128),
                    bias_t).reshape(_V_PAD, _D)
    return h[:_V]


# speedup vs baseline: 2.7318x; 1.1831x over previous
import functools

import jax
import jax.numpy as jnp
from jax import lax
from jax.experimental import pallas as pl
from jax.experimental.pallas import tpu as pltpu
from jax.experimental.pallas import tpu_sc as plsc

_V = 10000
_E = 160000
_NODE_IN = 128
_EDGE_IN = 16
_D = 32
_EH = 128
_STEPS = 6

_NC = 2
_NS = 16
_NW = _NC * _NS
_CH = 128
_CHUNKS = 40
_EPW = _CH * _CHUNKS
_E_PAD = _NW * _EPW
_V_PAD = 10240
_RPW = _V_PAD // _NS
_WCH = 8
_WAVES = _CHUNKS // _WCH
_WROWS = _WCH * _CH


def _ew_body(ef_ref, w1_ref, b1_ref, w2_ref, b2_ref, out_ref):
    a = jnp.maximum(
        jnp.dot(ef_ref[...], w1_ref[...], preferred_element_type=jnp.float32)
        + b1_ref[...], 0.0)
    out_ref[...] = (
        jnp.dot(a.astype(jnp.bfloat16), w2_ref[...],
                preferred_element_type=jnp.float32)
        + b2_ref[...]).astype(jnp.bfloat16)


def _proj_body(x_ref, w_ref, b_ref, out_ref):
    out_ref[...] = jnp.maximum(
        jnp.dot(x_ref[...], w_ref[...], preferred_element_type=jnp.float32)
        + b_ref[...], 0.0)


def _msg_body(hs_ref, ew_ref, r_ref, m_ref):
    hsb = hs_ref[...].astype(jnp.bfloat16)
    hst = pltpu.repeat(hsb, _D, axis=1)
    p = hst * ew_ref[...]
    m_ref[...] = jnp.dot(p, r_ref[...], preferred_element_type=jnp.float32)


def _combine_body(p_ref, b_ref, out_ref):
    out_ref[...] = jnp.maximum(p_ref[0] + p_ref[1] + b_ref[...], 0.0)


_MESH = plsc.VectorSubcoreMesh(core_axis_name="c", subcore_axis_name="s")
_SC_PARAMS = pltpu.CompilerParams(use_tc_tiling_on_sc=False)


@functools.partial(
    pl.kernel,
    mesh=_MESH,
    compiler_params=_SC_PARAMS,
    out_type=jax.ShapeDtypeStruct((_E_PAD, _D), jnp.float32),
    scratch_types=[
        pltpu.VMEM((_CHUNKS, _CH), jnp.int32),
        pltpu.VMEM((_WROWS, _D), jnp.float32),
        pltpu.VMEM((_WROWS, _D), jnp.float32),
        pltpu.VMEM_SHARED((_V_PAD, _D), jnp.float32),
        pltpu.SemaphoreType.DMA,
        pltpu.SemaphoreType.DMA,
    ],
)
def _gather(h_hbm, idx_hbm, hs_hbm, idx_v, buf0, buf1, h_sh, gsem, wsem):
    sid = lax.axis_index("s")
    wid = sid * _NC + lax.axis_index("c")
    stage = pltpu.async_copy(
        h_hbm.at[pl.ds(sid * _RPW, _RPW)], h_sh.at[pl.ds(sid * _RPW, _RPW)],
        wsem)
    pltpu.sync_copy(idx_hbm.at[wid], idx_v)
    stage.wait()
    plsc.subcore_barrier()
    base = wid * _EPW
    bufs = (buf0, buf1)
    wdesc = [None, None]
    for w in range(_WAVES):
        buf = bufs[w % 2]
        if wdesc[w % 2] is not None:
            wdesc[w % 2].wait()
        descs = []
        for j in range(_WCH):
            c = w * _WCH + j
            descs.append(pltpu.async_copy(
                h_sh.at[idx_v.at[c]], buf.at[pl.ds(j * _CH, _CH)], gsem))
        for dsc in descs:
            dsc.wait()
        wdesc[w % 2] = pltpu.async_copy(
            buf, hs_hbm.at[pl.ds(base + w * _WROWS, _WROWS)], wsem)
    for d in wdesc:
        if d is not None:
            d.wait()


@functools.partial(
    pl.kernel,
    mesh=_MESH,
    compiler_params=_SC_PARAMS,
    out_type=jax.ShapeDtypeStruct((_NC, _V_PAD, _D), jnp.float32),
    scratch_types=[
        pltpu.VMEM((_CHUNKS, _CH), jnp.int32),
        pltpu.VMEM((_WROWS, _D), jnp.float32),
        pltpu.VMEM((_WROWS, _D), jnp.float32),
        pltpu.VMEM((_RPW, _D), jnp.float32),
        pltpu.VMEM_SHARED((_V_PAD, _D), jnp.float32),
        pltpu.SemaphoreType.DMA,
        pltpu.SemaphoreType.DMA,
    ],
)
def _scatter(m_hbm, idx_hbm, zeros_hbm, out_hbm, idx_v, buf0, buf1, tbuf_v,
             acc_sh, lsem, ssem):
    cid = lax.axis_index("c")
    sid = lax.axis_index("s")
    wid = sid * _NC + cid
    base = wid * _EPW
    bufs = (buf0, buf1)
    pltpu.sync_copy(idx_hbm.at[wid], idx_v)
    pltpu.sync_copy(zeros_hbm.at[pl.ds(sid * _RPW, _RPW)], tbuf_v)
    pltpu.sync_copy(tbuf_v, acc_sh.at[pl.ds(sid * _RPW, _RPW)])

    ld = [None, None]
    ld[0] = pltpu.async_copy(m_hbm.at[pl.ds(base, _WROWS)], buf0, lsem)
    plsc.subcore_barrier()
    adds = [[], []]
    for w in range(_WAVES):
        p = w % 2
        if w + 1 < _WAVES:
            pn = (w + 1) % 2
            for d in adds[pn]:
                d.wait()
            adds[pn] = []
            ld[pn] = pltpu.async_copy(
                m_hbm.at[pl.ds(base + (w + 1) * _WROWS, _WROWS)],
                bufs[pn], lsem)
        ld[p].wait()
        for j in range(_WCH):
            adds[p].append(pltpu.async_copy(
                bufs[p].at[pl.ds(j * _CH, _CH)],
                acc_sh.at[idx_v.at[w * _WCH + j]], ssem, add=True))
    for par in adds:
        for d in par:
            d.wait()
    plsc.subcore_barrier()
    pltpu.sync_copy(acc_sh.at[pl.ds(sid * _RPW, _RPW)], tbuf_v)
    pltpu.sync_copy(tbuf_v, out_hbm.at[cid].at[pl.ds(sid * _RPW, _RPW)])


def kernel(node_feats, edge_feats, W_proj, b_proj, W1, b1, W2, b2, bias,
           edge_index):
    f32 = jnp.float32
    src = edge_index[0]
    dst = edge_index[1]

    cols = jnp.arange(_D * _D, dtype=jnp.int32)
    perm = (cols % _D) * _D + cols // _D
    W2p = W2[:, perm]
    b2p = b2[perm]

    ef_pad = jnp.zeros((_E_PAD, _EDGE_IN), dtype=f32).at[:_E].set(edge_feats)

    x_pad = jnp.zeros((_V_PAD, _NODE_IN), dtype=f32).at[:_V].set(node_feats)

    src_rs = jnp.concatenate(
        [src, jnp.zeros((_E_PAD - _E,), jnp.int32)]).reshape(_NW, _CHUNKS, _CH)
    dst_rs = jnp.concatenate(
        [dst, jnp.full((_E_PAD - _E,), _V, jnp.int32)]).reshape(
            _NW, _CHUNKS, _CH)

    r_mat = jnp.kron(jnp.eye(_D, dtype=jnp.bfloat16),
                     jnp.ones((_D, 1), dtype=jnp.bfloat16))
    zeros_acc = jnp.zeros((_V_PAD, _D), dtype=f32)

    be = 2048
    ew3 = pl.pallas_call(
        _ew_body,
        grid=(_E_PAD // be,),
        in_specs=[
            pl.BlockSpec((be, _EDGE_IN), lambda i: (i, 0)),
            pl.BlockSpec((_EDGE_IN, _EH), lambda i: (0, 0)),
            pl.BlockSpec((1, _EH), lambda i: (0, 0)),
            pl.BlockSpec((_EH, _D * _D), lambda i: (0, 0)),
            pl.BlockSpec((1, _D * _D), lambda i: (0, 0)),
        ],
        out_specs=pl.BlockSpec((be, _D * _D), lambda i: (i, 0)),
        out_shape=jax.ShapeDtypeStruct((_E_PAD, _D * _D), jnp.bfloat16),
    )(ef_pad, W1, b1.reshape(1, _EH), W2p.astype(jnp.bfloat16),
      b2p.reshape(1, _D * _D))

    h = pl.pallas_call(
        _proj_body,
        in_specs=[
            pl.BlockSpec((_V_PAD, _NODE_IN), lambda: (0, 0)),
            pl.BlockSpec((_NODE_IN, _D), lambda: (0, 0)),
            pl.BlockSpec((1, _D), lambda: (0, 0)),
        ],
        out_specs=pl.BlockSpec((_V_PAD, _D), lambda: (0, 0)),
        out_shape=jax.ShapeDtypeStruct((_V_PAD, _D), f32),
    )(x_pad, W_proj, b_proj.reshape(1, _D))

    bm = 1024
    msg = pl.pallas_call(
        _msg_body,
        grid=(_E_PAD // bm,),
        in_specs=[
            pl.BlockSpec((bm, _D), lambda i: (i, 0)),
            pl.BlockSpec((bm, _D * _D), lambda i: (i, 0)),
            pl.BlockSpec((_D * _D, _D), lambda i: (0, 0)),
        ],
        out_specs=pl.BlockSpec((bm, _D), lambda i: (i, 0)),
        out_shape=jax.ShapeDtypeStruct((_E_PAD, _D), f32),
    )

    combine = pl.pallas_call(
        _combine_body,
        in_specs=[
            pl.BlockSpec((_NC, _V_PAD, _D), lambda: (0, 0, 0)),
            pl.BlockSpec((1, _D), lambda: (0, 0)),
        ],
        out_specs=pl.BlockSpec((_V_PAD, _D), lambda: (0, 0)),
        out_shape=jax.ShapeDtypeStruct((_V_PAD, _D), f32),
    )

    bias_2d = bias.reshape(1, _D)
    for _ in range(_STEPS):
        hs = _gather(h, src_rs)
        m = msg(hs, ew3, r_mat)
        p = _scatter(m, dst_rs, zeros_acc)
        h = combine(p, bias_2d)
    return h[:_V]


# speedup vs baseline: 3.4737x; 1.2716x over previous
import functools

import jax
import jax.numpy as jnp
from jax import lax
from jax.experimental import pallas as pl
from jax.experimental.pallas import tpu as pltpu
from jax.experimental.pallas import tpu_sc as plsc

_V = 10000
_E = 160000
_NODE_IN = 128
_EDGE_IN = 16
_D = 32
_EH = 128
_STEPS = 6

_NC = 2
_NS = 16
_NW = _NC * _NS
_CH = 128
_CHUNKS = 40
_EPW = _CH * _CHUNKS
_E_PAD = _NW * _EPW
_V_PAD = 10240
_RPW = _V_PAD // _NS
_WCH = 8
_WAVES = _CHUNKS // _WCH
_WROWS = _WCH * _CH


def _ew_body(ef_ref, w1_ref, b1_ref, w2_ref, b2_ref, out_ref):
    a = jnp.maximum(
        jnp.dot(ef_ref[...], w1_ref[...], preferred_element_type=jnp.float32)
        + b1_ref[...], 0.0)
    out_ref[...] = (
        jnp.dot(a.astype(jnp.bfloat16), w2_ref[...],
                preferred_element_type=jnp.float32)
        + b2_ref[...]).astype(jnp.bfloat16)


def _proj_body(x_ref, w_ref, b_ref, out_ref):
    out_ref[...] = jnp.maximum(
        jnp.dot(x_ref[...], w_ref[...], preferred_element_type=jnp.float32)
        + b_ref[...], 0.0)


def _msg_body(hs_ref, ew_ref, r_ref, m_ref):
    g = hs_ref.shape[0]
    hsw = hs_ref[...].astype(jnp.bfloat16)
    eww = ew_ref[...]
    outs = []
    for j in range(4):
        hj = hsw[:, j * _D:(j + 1) * _D]
        hrep = pltpu.repeat(hj, _D, axis=1)
        pj = hrep * eww[j * g:(j + 1) * g, :]
        outs.append(jnp.dot(pj, r_ref[...],
                            preferred_element_type=jnp.float32))
    m_ref[...] = jnp.concatenate(outs, axis=1)


def _combine_body(p_ref, b_ref, out_ref):
    out_ref[...] = jnp.maximum(p_ref[0] + p_ref[1] + b_ref[...], 0.0)


_MESH = plsc.VectorSubcoreMesh(core_axis_name="c", subcore_axis_name="s")
_SC_PARAMS = pltpu.CompilerParams(use_tc_tiling_on_sc=False)


@functools.partial(
    pl.kernel,
    mesh=_MESH,
    compiler_params=_SC_PARAMS,
    out_type=jax.ShapeDtypeStruct((_E_PAD, _D), jnp.float32),
    scratch_types=[
        pltpu.VMEM((_CHUNKS, _CH), jnp.int32),
        pltpu.VMEM((_WROWS, _D), jnp.float32),
        pltpu.VMEM((_WROWS, _D), jnp.float32),
        pltpu.VMEM_SHARED((_V_PAD, _D), jnp.float32),
        pltpu.SemaphoreType.DMA,
        pltpu.SemaphoreType.DMA,
    ],
)
def _gather(h_hbm, idx_hbm, hs_hbm, idx_v, buf0, buf1, h_sh, gsem, wsem):
    sid = lax.axis_index("s")
    wid = sid * _NC + lax.axis_index("c")
    stage = pltpu.async_copy(
        h_hbm.at[pl.ds(sid * _RPW, _RPW)], h_sh.at[pl.ds(sid * _RPW, _RPW)],
        wsem)
    pltpu.sync_copy(idx_hbm.at[wid], idx_v)
    stage.wait()
    plsc.subcore_barrier()
    base = wid * _EPW
    bufs = (buf0, buf1)
    wdesc = [None, None]
    for w in range(_WAVES):
        buf = bufs[w % 2]
        if wdesc[w % 2] is not None:
            wdesc[w % 2].wait()
        descs = []
        for j in range(_WCH):
            c = w * _WCH + j
            descs.append(pltpu.async_copy(
                h_sh.at[idx_v.at[c]], buf.at[pl.ds(j * _CH, _CH)], gsem))
        for dsc in descs:
            dsc.wait()
        wdesc[w % 2] = pltpu.async_copy(
            buf, hs_hbm.at[pl.ds(base + w * _WROWS, _WROWS)], wsem)
    for d in wdesc:
        if d is not None:
            d.wait()


@functools.partial(
    pl.kernel,
    mesh=_MESH,
    compiler_params=_SC_PARAMS,
    out_type=jax.ShapeDtypeStruct((_NC, _V_PAD, _D), jnp.float32),
    scratch_types=[
        pltpu.VMEM((_CHUNKS, _CH), jnp.int32),
        pltpu.VMEM((_WROWS, _D), jnp.float32),
        pltpu.VMEM((_WROWS, _D), jnp.float32),
        pltpu.VMEM((_RPW, _D), jnp.float32),
        pltpu.VMEM_SHARED((_V_PAD, _D), jnp.float32),
        pltpu.SemaphoreType.DMA,
        pltpu.SemaphoreType.DMA,
    ],
)
def _scatter(m_hbm, idx_hbm, zeros_hbm, out_hbm, idx_v, buf0, buf1, tbuf_v,
             acc_sh, lsem, ssem):
    cid = lax.axis_index("c")
    sid = lax.axis_index("s")
    wid = sid * _NC + cid
    base = wid * _EPW
    bufs = (buf0, buf1)
    pltpu.sync_copy(idx_hbm.at[wid], idx_v)
    pltpu.sync_copy(zeros_hbm.at[pl.ds(sid * _RPW, _RPW)], tbuf_v)
    pltpu.sync_copy(tbuf_v, acc_sh.at[pl.ds(sid * _RPW, _RPW)])

    ld = [None, None]
    ld[0] = pltpu.async_copy(m_hbm.at[pl.ds(base, _WROWS)], buf0, lsem)
    plsc.subcore_barrier()
    adds = [[], []]
    for w in range(_WAVES):
        p = w % 2
        if w + 1 < _WAVES:
            pn = (w + 1) % 2
            for d in adds[pn]:
                d.wait()
            adds[pn] = []
            ld[pn] = pltpu.async_copy(
                m_hbm.at[pl.ds(base + (w + 1) * _WROWS, _WROWS)],
                bufs[pn], lsem)
        ld[p].wait()
        for j in range(_WCH):
            adds[p].append(pltpu.async_copy(
                bufs[p].at[pl.ds(j * _CH, _CH)],
                acc_sh.at[idx_v.at[w * _WCH + j]], ssem, add=True))
    for par in adds:
        for d in par:
            d.wait()
    plsc.subcore_barrier()
    pltpu.sync_copy(acc_sh.at[pl.ds(sid * _RPW, _RPW)], tbuf_v)
    pltpu.sync_copy(tbuf_v, out_hbm.at[cid].at[pl.ds(sid * _RPW, _RPW)])


def kernel(node_feats, edge_feats, W_proj, b_proj, W1, b1, W2, b2, bias,
           edge_index):
    f32 = jnp.float32
    src = edge_index[0]
    dst = edge_index[1]

    cols = jnp.arange(_D * _D, dtype=jnp.int32)
    perm = (cols % _D) * _D + cols // _D
    W2p = W2[:, perm]
    b2p = b2[perm]

    r_idx = jnp.arange(_E_PAD, dtype=jnp.int32)
    w_in = r_idx % 1024
    q_idx = (r_idx - w_in) + 4 * (w_in % 256) + w_in // 256
    ef_re = jnp.take(edge_feats, jnp.minimum(q_idx, _E - 1), axis=0)


    src_rs = jnp.concatenate(
        [src, jnp.zeros((_E_PAD - _E,), jnp.int32)]).reshape(_NW, _CHUNKS, _CH)
    dst_rs = jnp.concatenate(
        [dst, jnp.full((_E_PAD - _E,), _V, jnp.int32)]).reshape(
            _NW, _CHUNKS, _CH)

    r_mat = jnp.kron(jnp.eye(_D, dtype=jnp.bfloat16),
                     jnp.ones((_D, 1), dtype=jnp.bfloat16))
    zeros_acc = jnp.zeros((_V_PAD, _D), dtype=f32)

    be = 2048
    ew3 = pl.pallas_call(
        _ew_body,
        grid=(_E_PAD // be,),
        in_specs=[
            pl.BlockSpec((be, _EDGE_IN), lambda i: (i, 0)),
            pl.BlockSpec((_EDGE_IN, _EH), lambda i: (0, 0)),
            pl.BlockSpec((1, _EH), lambda i: (0, 0)),
            pl.BlockSpec((_EH, _D * _D), lambda i: (0, 0)),
            pl.BlockSpec((1, _D * _D), lambda i: (0, 0)),
        ],
        out_specs=pl.BlockSpec((be, _D * _D), lambda i: (i, 0)),
        out_shape=jax.ShapeDtypeStruct((_E_PAD, _D * _D), jnp.bfloat16),
    )(ef_re, W1, b1.reshape(1, _EH), W2p.astype(jnp.bfloat16),
      b2p.reshape(1, _D * _D))

    h = pl.pallas_call(
        _proj_body,
        grid=(1,),
        in_specs=[
            pl.BlockSpec((_V, _NODE_IN), lambda i: (0, 0)),
            pl.BlockSpec((_NODE_IN, _D), lambda i: (0, 0)),
            pl.BlockSpec((1, _D), lambda i: (0, 0)),
        ],
        out_specs=pl.BlockSpec((_V, _D), lambda i: (0, 0)),
        out_shape=jax.ShapeDtypeStruct((_V_PAD, _D), f32),
    )(node_feats, W_proj, b_proj.reshape(1, _D))

    bm = 1024
    msg = pl.pallas_call(
        _msg_body,
        grid=(_E_PAD // bm,),
        in_specs=[
            pl.BlockSpec((bm // 4, _D * 4), lambda i: (i, 0)),
            pl.BlockSpec((bm, _D * _D), lambda i: (i, 0)),
            pl.BlockSpec((_D * _D, _D), lambda i: (0, 0)),
        ],
        out_specs=pl.BlockSpec((bm // 4, _D * 4), lambda i: (i, 0)),
        out_shape=jax.ShapeDtypeStruct((_E_PAD // 4, _D * 4), f32),
    )

    vq = _V_PAD // 4
    combine = pl.pallas_call(
        _combine_body,
        grid=(1,),
        in_specs=[
            pl.BlockSpec((_NC, vq, 128), lambda i: (0, 0, 0)),
            pl.BlockSpec((1, 128), lambda i: (0, 0)),
        ],
        out_specs=pl.BlockSpec((vq, 128), lambda i: (0, 0)),
        out_shape=jax.ShapeDtypeStruct((vq, 128), f32),
    )

    bias_t = jnp.tile(bias, 4).reshape(1, 128)
    for _ in range(_STEPS):
        hs = _gather(h, src_rs)
        m_w = msg(hs.reshape(_E_PAD // 4, _D * 4), ew3, r_mat)
        p = _scatter(m_w.reshape(_E_PAD, _D), dst_rs, zeros_acc)
        h = combine(p.reshape(_NC, vq, 128), bias_t).reshape(_V_PAD, _D)
    return h[:_V]


# speedup vs baseline: 4.0438x; 1.1641x over previous
import functools

import jax
import jax.numpy as jnp
from jax import lax
from jax.experimental import pallas as pl
from jax.experimental.pallas import tpu as pltpu
from jax.experimental.pallas import tpu_sc as plsc

_V = 10000
_E = 160000
_NODE_IN = 128
_EDGE_IN = 16
_D = 32
_EH = 128
_STEPS = 6

_NC = 2
_NS = 16
_NW = _NC * _NS
_CH = 128
_CHUNKS = 40
_EPW = _CH * _CHUNKS
_E_PAD = _NW * _EPW
_V_PAD = 10240
_RPW = _V_PAD // _NS
_WCH = 8
_WAVES = _CHUNKS // _WCH
_WROWS = _WCH * _CH


def _ew_body(ef_ref, w1_ref, b1_ref, w2_ref, b2_ref, out_ref):
    a = jnp.maximum(
        jnp.dot(ef_ref[...], w1_ref[...], preferred_element_type=jnp.float32)
        + b1_ref[...], 0.0)
    out_ref[...] = (
        jnp.dot(a.astype(jnp.bfloat16), w2_ref[...],
                preferred_element_type=jnp.float32)
        + b2_ref[...]).astype(jnp.bfloat16)


def _proj_body(x_ref, w_ref, b_ref, out_ref):
    out_ref[...] = jnp.maximum(
        jnp.dot(x_ref[...], w_ref[...], preferred_element_type=jnp.float32)
        + b_ref[...], 0.0)


def _msg_body(hs_ref, ew_ref, r_ref, m_ref):
    g = hs_ref.shape[0]
    hsw = hs_ref[...].astype(jnp.bfloat16)
    eww = ew_ref[...]
    outs = []
    for j in range(4):
        hj = hsw[:, j * _D:(j + 1) * _D]
        hrep = pltpu.repeat(hj, _D, axis=1)
        pj = hrep * eww[j * g:(j + 1) * g, :]
        outs.append(jnp.dot(pj, r_ref[...],
                            preferred_element_type=jnp.float32))
    m_ref[...] = jnp.concatenate(outs, axis=1)


def _combine_body(p_ref, b_ref, out_ref):
    out_ref[...] = jnp.maximum(p_ref[0] + p_ref[1] + b_ref[...], 0.0)


_MESH = plsc.VectorSubcoreMesh(core_axis_name="c", subcore_axis_name="s")
_SC_PARAMS = pltpu.CompilerParams(use_tc_tiling_on_sc=False)


@functools.partial(
    pl.kernel,
    mesh=_MESH,
    compiler_params=_SC_PARAMS,
    out_type=jax.ShapeDtypeStruct((_E_PAD, _D), jnp.float32),
    scratch_types=[
        pltpu.VMEM((_CHUNKS, _CH), jnp.int32),
        pltpu.VMEM((_WROWS, _D), jnp.float32),
        pltpu.VMEM((_WROWS, _D), jnp.float32),
        pltpu.VMEM_SHARED((_V_PAD, _D), jnp.float32),
        pltpu.SemaphoreType.DMA,
        pltpu.SemaphoreType.DMA,
    ],
)
def _gather(h_hbm, idx_hbm, hs_hbm, idx_v, buf0, buf1, h_sh, gsem, wsem):
    sid = lax.axis_index("s")
    wid = sid * _NC + lax.axis_index("c")
    stage = pltpu.async_copy(
        h_hbm.at[pl.ds(sid * _RPW, _RPW)], h_sh.at[pl.ds(sid * _RPW, _RPW)],
        wsem)
    pltpu.sync_copy(idx_hbm.at[wid], idx_v)
    stage.wait()
    plsc.subcore_barrier()
    base = wid * _EPW
    bufs = (buf0, buf1)
    wdesc = [None, None]
    for w in range(_WAVES):
        buf = bufs[w % 2]
        if wdesc[w % 2] is not None:
            wdesc[w % 2].wait()
        descs = []
        for j in range(_WCH):
            c = w * _WCH + j
            descs.append(pltpu.async_copy(
                h_sh.at[idx_v.at[c]], buf.at[pl.ds(j * _CH, _CH)], gsem))
        for dsc in descs:
            dsc.wait()
        wdesc[w % 2] = pltpu.async_copy(
            buf, hs_hbm.at[pl.ds(base + w * _WROWS, _WROWS)], wsem)
    for d in wdesc:
        if d is not None:
            d.wait()


@functools.partial(
    pl.kernel,
    mesh=_MESH,
    compiler_params=_SC_PARAMS,
    out_type=jax.ShapeDtypeStruct((_NC, _V_PAD, _D), jnp.float32),
    scratch_types=[
        pltpu.VMEM((_CHUNKS, _CH), jnp.int32),
        pltpu.VMEM((_WROWS, _D), jnp.float32),
        pltpu.VMEM((_WROWS, _D), jnp.float32),
        pltpu.VMEM((_RPW, _D), jnp.float32),
        pltpu.VMEM_SHARED((_V_PAD, _D), jnp.float32),
        pltpu.SemaphoreType.DMA,
        pltpu.SemaphoreType.DMA,
    ],
)
def _scatter(m_hbm, idx_hbm, zeros_hbm, out_hbm, idx_v, buf0, buf1, tbuf_v,
             acc_sh, lsem, ssem):
    cid = lax.axis_index("c")
    sid = lax.axis_index("s")
    wid = sid * _NC + cid
    base = wid * _EPW
    bufs = (buf0, buf1)
    pltpu.sync_copy(idx_hbm.at[wid], idx_v)
    pltpu.sync_copy(zeros_hbm.at[pl.ds(sid * _RPW, _RPW)], tbuf_v)
    pltpu.sync_copy(tbuf_v, acc_sh.at[pl.ds(sid * _RPW, _RPW)])

    ld = [None, None]
    ld[0] = pltpu.async_copy(m_hbm.at[pl.ds(base, _WROWS)], buf0, lsem)
    plsc.subcore_barrier()
    adds = [[], []]
    for w in range(_WAVES):
        p = w % 2
        if w + 1 < _WAVES:
            pn = (w + 1) % 2
            for d in adds[pn]:
                d.wait()
            adds[pn] = []
            ld[pn] = pltpu.async_copy(
                m_hbm.at[pl.ds(base + (w + 1) * _WROWS, _WROWS)],
                bufs[pn], lsem)
        ld[p].wait()
        for j in range(_WCH):
            adds[p].append(pltpu.async_copy(
                bufs[p].at[pl.ds(j * _CH, _CH)],
                acc_sh.at[idx_v.at[w * _WCH + j]], ssem, add=True))
    for par in adds:
        for d in par:
            d.wait()
    plsc.subcore_barrier()
    pltpu.sync_copy(acc_sh.at[pl.ds(sid * _RPW, _RPW)], tbuf_v)
    pltpu.sync_copy(tbuf_v, out_hbm.at[cid].at[pl.ds(sid * _RPW, _RPW)])


def kernel(node_feats, edge_feats, W_proj, b_proj, W1, b1, W2, b2, bias,
           edge_index):
    f32 = jnp.float32
    src = edge_index[0]
    dst = edge_index[1]

    cols = jnp.arange(_D * _D, dtype=jnp.int32)
    perm = (cols % _D) * _D + cols // _D
    W2p = W2[:, perm]
    b2p = b2[perm]

    p_idx = jnp.arange(_E_PAD, dtype=jnp.int32)
    w_in = p_idx % 1024
    psi = (p_idx - w_in) + (w_in % 4) * 256 + w_in // 4
    psi_c = jnp.minimum(psi, _E - 1)


    src_rs = jnp.take(src, psi_c).reshape(_NW, _CHUNKS, _CH)
    dst_rs = jnp.where(psi >= _E, _V, jnp.take(dst, psi_c)).reshape(
        _NW, _CHUNKS, _CH)

    r_mat = jnp.kron(jnp.eye(_D, dtype=jnp.bfloat16),
                     jnp.ones((_D, 1), dtype=jnp.bfloat16))
    zeros_acc = jnp.zeros((_V_PAD, _D), dtype=f32)

    be = 2000
    ew3 = pl.pallas_call(
        _ew_body,
        grid=(_E // be,),
        in_specs=[
            pl.BlockSpec((be, _EDGE_IN), lambda i: (i, 0)),
            pl.BlockSpec((_EDGE_IN, _EH), lambda i: (0, 0)),
            pl.BlockSpec((1, _EH), lambda i: (0, 0)),
            pl.BlockSpec((_EH, _D * _D), lambda i: (0, 0)),
            pl.BlockSpec((1, _D * _D), lambda i: (0, 0)),
        ],
        out_specs=pl.BlockSpec((be, _D * _D), lambda i: (i, 0)),
        out_shape=jax.ShapeDtypeStruct((_E_PAD, _D * _D), jnp.bfloat16),
    )(edge_feats, W1, b1.reshape(1, _EH), W2p.astype(jnp.bfloat16),
      b2p.reshape(1, _D * _D))

    h = pl.pallas_call(
        _proj_body,
        grid=(1,),
        in_specs=[
            pl.BlockSpec((_V, _NODE_IN), lambda i: (0, 0)),
            pl.BlockSpec((_NODE_IN, _D), lambda i: (0, 0)),
            pl.BlockSpec((1, _D), lambda i: (0, 0)),
        ],
        out_specs=pl.BlockSpec((_V, _D), lambda i: (0, 0)),
        out_shape=jax.ShapeDtypeStruct((_V_PAD, _D), f32),
    )(node_feats, W_proj, b_proj.reshape(1, _D))

    bm = 1024
    msg = pl.pallas_call(
        _msg_body,
        grid=(_E_PAD // bm,),
        in_specs=[
            pl.BlockSpec((bm // 4, _D * 4), lambda i: (i, 0)),
            pl.BlockSpec((bm, _D * _D), lambda i: (i, 0)),
            pl.BlockSpec((_D * _D, _D), lambda i: (0, 0)),
        ],
        out_specs=pl.BlockSpec((bm // 4, _D * 4), lambda i: (i, 0)),
        out_shape=jax.ShapeDtypeStruct((_E_PAD // 4, _D * 4), f32),
    )

    vq = _V_PAD // 4
    combine = pl.pallas_call(
        _combine_body,
        grid=(1,),
        in_specs=[
            pl.BlockSpec((_NC, vq, 128), lambda i: (0, 0, 0)),
            pl.BlockSpec((1, 128), lambda i: (0, 0)),
        ],
        out_specs=pl.BlockSpec((vq, 128), lambda i: (0, 0)),
        out_shape=jax.ShapeDtypeStruct((vq, 128), f32),
    )

    bias_t = jnp.tile(bias, 4).reshape(1, 128)
    for _ in range(_STEPS):
        hs = _gather(h, src_rs)
        m_w = msg(hs.reshape(_E_PAD // 4, _D * 4), ew3, r_mat)
        p = _scatter(m_w.reshape(_E_PAD, _D), dst_rs, zeros_acc)
        h = combine(p.reshape(_NC, vq, 128), bias_t).reshape(_V_PAD, _D)
    return h[:_V]


# speedup vs baseline: 4.7787x; 1.1817x over previous
import functools

import jax
import jax.numpy as jnp
from jax import lax
from jax.experimental import pallas as pl
from jax.experimental.pallas import tpu as pltpu
from jax.experimental.pallas import tpu_sc as plsc

_V = 10000
_E = 160000
_NODE_IN = 128
_EDGE_IN = 16
_D = 32
_EH = 128
_STEPS = 6

_NC = 2
_NS = 16
_NW = _NC * _NS
_CH = 128
_CHUNKS = 40
_EPW = _CH * _CHUNKS
_E_PAD = _NW * _EPW
_V_PAD = 10240
_RPW = _V_PAD // _NS
_WCH = 8
_WAVES = _CHUNKS // _WCH
_WROWS = _WCH * _CH
_BM = 2048
_G = _BM // 4


def _ew_body(ef_ref, w1_ref, b1_ref, w2_ref, b2_ref, out_ref):
    a = jnp.maximum(
        jnp.dot(ef_ref[...], w1_ref[...], preferred_element_type=jnp.float32)
        + b1_ref[...], 0.0)
    out_ref[...] = (
        jnp.dot(a.astype(jnp.bfloat16), w2_ref[...],
                preferred_element_type=jnp.float32)
        + b2_ref[...]).astype(jnp.bfloat16)


def _proj_body(x_ref, w_ref, b_ref, out_ref):
    out_ref[...] = jnp.maximum(
        jnp.dot(x_ref[...], w_ref[...], preferred_element_type=jnp.float32)
        + b_ref[...], 0.0)


def _msg_body(hs_ref, ew_ref, r_ref, m_ref):
    g = hs_ref.shape[0]
    hsw = hs_ref[...].astype(jnp.bfloat16)
    eww = ew_ref[...]
    outs = []
    for j in range(4):
        hj = hsw[:, j * _D:(j + 1) * _D]
        hrep = pltpu.repeat(hj, _D, axis=1)
        pj = hrep * eww[j * g:(j + 1) * g, :]
        outs.append(jnp.dot(pj, r_ref[...],
                            preferred_element_type=jnp.float32))
    m_ref[...] = jnp.concatenate(outs, axis=1)


def _combine_body(p_ref, b_ref, out_ref):
    out_ref[...] = jnp.maximum(p_ref[0] + p_ref[1] + b_ref[...], 0.0)


_MESH = plsc.VectorSubcoreMesh(core_axis_name="c", subcore_axis_name="s")
_SC_PARAMS = pltpu.CompilerParams(use_tc_tiling_on_sc=False)


@functools.partial(
    pl.kernel,
    mesh=_MESH,
    compiler_params=_SC_PARAMS,
    out_type=jax.ShapeDtypeStruct((_E_PAD, _D), jnp.float32),
    scratch_types=[
        pltpu.VMEM((_CHUNKS, _CH), jnp.int32),
        pltpu.VMEM((_WROWS, _D), jnp.float32),
        pltpu.VMEM((_WROWS, _D), jnp.float32),
        pltpu.VMEM_SHARED((_V_PAD, _D), jnp.float32),
        pltpu.SemaphoreType.DMA,
        pltpu.SemaphoreType.DMA,
    ],
)
def _gather(h_hbm, idx_hbm, hs_hbm, idx_v, buf0, buf1, h_sh, gsem, wsem):
    sid = lax.axis_index("s")
    wid = sid * _NC + lax.axis_index("c")
    stage = pltpu.async_copy(
        h_hbm.at[pl.ds(sid * _RPW, _RPW)], h_sh.at[pl.ds(sid * _RPW, _RPW)],
        wsem)
    pltpu.sync_copy(idx_hbm.at[wid], idx_v)
    stage.wait()
    plsc.subcore_barrier()
    base = wid * _EPW
    bufs = (buf0, buf1)
    wdesc = [None, None]
    for w in range(_WAVES):
        buf = bufs[w % 2]
        if wdesc[w % 2] is not None:
            wdesc[w % 2].wait()
        descs = []
        for j in range(_WCH):
            c = w * _WCH + j
            descs.append(pltpu.async_copy(
                h_sh.at[idx_v.at[c]], buf.at[pl.ds(j * _CH, _CH)], gsem))
        for dsc in descs:
            dsc.wait()
        wdesc[w % 2] = pltpu.async_copy(
            buf, hs_hbm.at[pl.ds(base + w * _WROWS, _WROWS)], wsem)
    for d in wdesc:
        if d is not None:
            d.wait()


@functools.partial(
    pl.kernel,
    mesh=_MESH,
    compiler_params=_SC_PARAMS,
    out_type=jax.ShapeDtypeStruct((_E_PAD, _D), jnp.float32),
    scratch_types=[
        pltpu.VMEM((_CHUNKS, _CH), jnp.int32),
        pltpu.VMEM((_WROWS, _D), jnp.float32),
        pltpu.VMEM((_WROWS, _D), jnp.float32),
        pltpu.VMEM((_RPW, _D), jnp.float32),
        pltpu.VMEM((2, 16), jnp.float32),
        pltpu.VMEM_SHARED((_V_PAD, _D), jnp.float32),
        pltpu.SemaphoreType.DMA,
        pltpu.SemaphoreType.DMA,
    ],
)
def _gather_fused(p_hbm, bias_hbm, idx_hbm, hs_hbm, idx_v, buf0, buf1,
                  pb0, bbuf, h_sh, gsem, wsem):
    sid = lax.axis_index("s")
    wid = sid * _NC + lax.axis_index("c")
    pltpu.sync_copy(p_hbm.at[0].at[pl.ds(sid * _RPW, _RPW)], pb0)
    pltpu.sync_copy(p_hbm.at[1].at[pl.ds(sid * _RPW, _RPW)],
                    buf0.at[pl.ds(0, _RPW)])
    pltpu.sync_copy(bias_hbm, bbuf)
    pltpu.sync_copy(idx_hbm.at[wid], idx_v)
    blo = bbuf[0, :]
    bhi = bbuf[1, :]

    def body(r, carry):
        v0 = jnp.maximum(pb0[r, pl.ds(0, 16)] + buf0[r, pl.ds(0, 16)] + blo,
                         0.0)
        pb0[r, pl.ds(0, 16)] = v0
        v1 = jnp.maximum(pb0[r, pl.ds(16, 16)] + buf0[r, pl.ds(16, 16)] + bhi,
                         0.0)
        pb0[r, pl.ds(16, 16)] = v1
        return carry

    lax.fori_loop(0, _RPW, body, 0)
    pltpu.sync_copy(pb0, h_sh.at[pl.ds(sid * _RPW, _RPW)])
    plsc.subcore_barrier()
    base = wid * _EPW
    bufs = (buf0, buf1)
    wdesc = [None, None]
    for w in range(_WAVES):
        buf = bufs[w % 2]
        if wdesc[w % 2] is not None:
            wdesc[w % 2].wait()
        descs = []
        for j in range(_WCH):
            c = w * _WCH + j
            descs.append(pltpu.async_copy(
                h_sh.at[idx_v.at[c]], buf.at[pl.ds(j * _CH, _CH)], gsem))
        for dsc in descs:
            dsc.wait()
        wdesc[w % 2] = pltpu.async_copy(
            buf, hs_hbm.at[pl.ds(base + w * _WROWS, _WROWS)], wsem)
    for d in wdesc:
        if d is not None:
            d.wait()


@functools.partial(
    pl.kernel,
    mesh=_MESH,
    compiler_params=_SC_PARAMS,
    out_type=jax.ShapeDtypeStruct((_NC, _V_PAD, _D), jnp.float32),
    scratch_types=[
        pltpu.VMEM((_CHUNKS, _CH), jnp.int32),
        pltpu.VMEM((_WROWS, _D), jnp.float32),
        pltpu.VMEM((_WROWS, _D), jnp.float32),
        pltpu.VMEM((_RPW, _D), jnp.float32),
        pltpu.VMEM_SHARED((_V_PAD, _D), jnp.float32),
        pltpu.SemaphoreType.DMA,
        pltpu.SemaphoreType.DMA,
    ],
)
def _scatter(m_hbm, idx_hbm, zeros_hbm, out_hbm, idx_v, buf0, buf1, tbuf_v,
             acc_sh, lsem, ssem):
    cid = lax.axis_index("c")
    sid = lax.axis_index("s")
    wid = sid * _NC + cid
    base = wid * _EPW
    bufs = (buf0, buf1)
    pltpu.sync_copy(idx_hbm.at[wid], idx_v)
    pltpu.sync_copy(zeros_hbm.at[pl.ds(sid * _RPW, _RPW)], tbuf_v)
    pltpu.sync_copy(tbuf_v, acc_sh.at[pl.ds(sid * _RPW, _RPW)])

    ld = [None, None]
    ld[0] = pltpu.async_copy(m_hbm.at[pl.ds(base, _WROWS)], buf0, lsem)
    plsc.subcore_barrier()
    adds = [[], []]
    for w in range(_WAVES):
        p = w % 2
        if w + 1 < _WAVES:
            pn = (w + 1) % 2
            for d in adds[pn]:
                d.wait()
            adds[pn] = []
            ld[pn] = pltpu.async_copy(
                m_hbm.at[pl.ds(base + (w + 1) * _WROWS, _WROWS)],
                bufs[pn], lsem)
        ld[p].wait()
        for j in range(_WCH):
            adds[p].append(pltpu.async_copy(
                bufs[p].at[pl.ds(j * _CH, _CH)],
                acc_sh.at[idx_v.at[w * _WCH + j]], ssem, add=True))
    for par in adds:
        for d in par:
            d.wait()
    plsc.subcore_barrier()
    pltpu.sync_copy(acc_sh.at[pl.ds(sid * _RPW, _RPW)], tbuf_v)
    pltpu.sync_copy(tbuf_v, out_hbm.at[cid].at[pl.ds(sid * _RPW, _RPW)])


def kernel(node_feats, edge_feats, W_proj, b_proj, W1, b1, W2, b2, bias,
           edge_index):
    f32 = jnp.float32
    src = edge_index[0]
    dst = edge_index[1]

    cols = jnp.arange(_D * _D, dtype=jnp.int32)
    perm = (cols % _D) * _D + cols // _D
    W2p = W2[:, perm]
    b2p = b2[perm]

    p_idx = jnp.arange(_E_PAD, dtype=jnp.int32)
    w_in = p_idx % _BM
    psi = (p_idx - w_in) + (w_in % 4) * _G + w_in // 4
    psi_c = jnp.minimum(psi, _E - 1)


    src_rs = jnp.take(src, psi_c).reshape(_NW, _CHUNKS, _CH)
    dst_rs = jnp.where(psi >= _E, _V, jnp.take(dst, psi_c)).reshape(
        _NW, _CHUNKS, _CH)

    r_mat = jnp.kron(jnp.eye(_D, dtype=jnp.bfloat16),
                     jnp.ones((_D, 1), dtype=jnp.bfloat16))
    zeros_acc = jnp.zeros((_V_PAD, _D), dtype=f32)

    be = 2000
    ew3 = pl.pallas_call(
        _ew_body,
        grid=(_E // be,),
        in_specs=[
            pl.BlockSpec((be, _EDGE_IN), lambda i: (i, 0)),
            pl.BlockSpec((_EDGE_IN, _EH), lambda i: (0, 0)),
            pl.BlockSpec((1, _EH), lambda i: (0, 0)),
            pl.BlockSpec((_EH, _D * _D), lambda i: (0, 0)),
            pl.BlockSpec((1, _D * _D), lambda i: (0, 0)),
        ],
        out_specs=pl.BlockSpec((be, _D * _D), lambda i: (i, 0)),
        out_shape=jax.ShapeDtypeStruct((_E_PAD, _D * _D), jnp.bfloat16),
    )(edge_feats, W1, b1.reshape(1, _EH), W2p.astype(jnp.bfloat16),
      b2p.reshape(1, _D * _D))

    h = pl.pallas_call(
        _proj_body,
        grid=(1,),
        in_specs=[
            pl.BlockSpec((_V, _NODE_IN), lambda i: (0, 0)),
            pl.BlockSpec((_NODE_IN, _D), lambda i: (0, 0)),
            pl.BlockSpec((1, _D), lambda i: (0, 0)),
        ],
        out_specs=pl.BlockSpec((_V, _D), lambda i: (0, 0)),
        out_shape=jax.ShapeDtypeStruct((_V_PAD, _D), f32),
    )(node_feats, W_proj, b_proj.reshape(1, _D))

    msg = pl.pallas_call(
        _msg_body,
        grid=(_E_PAD // _BM,),
        in_specs=[
            pl.BlockSpec((_G, _D * 4), lambda i: (i, 0)),
            pl.BlockSpec((_BM, _D * _D), lambda i: (i, 0)),
            pl.BlockSpec((_D * _D, _D), lambda i: (0, 0)),
        ],
        out_specs=pl.BlockSpec((_G, _D * 4), lambda i: (i, 0)),
        out_shape=jax.ShapeDtypeStruct((_E_PAD // 4, _D * 4), f32),
    )

    vq = _V_PAD // 4
    combine = pl.pallas_call(
        _combine_body,
        grid=(1,),
        in_specs=[
            pl.BlockSpec((_NC, vq, 128), lambda i: (0, 0, 0)),
            pl.BlockSpec((1, 128), lambda i: (0, 0)),
        ],
        out_specs=pl.BlockSpec((vq, 128), lambda i: (0, 0)),
        out_shape=jax.ShapeDtypeStruct((vq, 128), f32),
    )

    bias_t = jnp.tile(bias, 4).reshape(1, 128)
    bias_2x16 = bias.reshape(2, 16)
    p = None
    for s in range(_STEPS):
        if s == 0:
            hs = _gather(h, src_rs)
        else:
            hs = _gather_fused(p, bias_2x16, src_rs)
        m_w = msg(hs.reshape(_E_PAD // 4, _D * 4), ew3, r_mat)
        p = _scatter(m_w.reshape(_E_PAD, _D), dst_rs, zeros_acc)
    h = combine(p.reshape(_NC, vq, 128), bias_t).reshape(_V_PAD, _D)
    return h[:_V]


# speedup vs baseline: 4.8366x; 1.0121x over previous
import functools

import jax
import jax.numpy as jnp
from jax import lax
from jax.experimental import pallas as pl
from jax.experimental.pallas import tpu as pltpu
from jax.experimental.pallas import tpu_sc as plsc

_V = 10000
_E = 160000
_NODE_IN = 128
_EDGE_IN = 16
_D = 32
_EH = 128
_STEPS = 6

_NC = 2
_NS = 16
_NW = _NC * _NS
_CH = 128
_CHUNKS = 40
_EPW = _CH * _CHUNKS
_E_PAD = _NW * _EPW
_V_PAD = 10240
_RPW = _V_PAD // _NS
_WCH = 8
_WAVES = _CHUNKS // _WCH
_WROWS = _WCH * _CH
_BM = 2048
_G = _BM // 4


def _ew_body(eft_ref, w1_ref, b1_ref, w2_ref, b2_ref, out_ref):
    a = jnp.maximum(
        jax.lax.dot_general(eft_ref[...], w1_ref[...], (((0,), (0,)), ((), ())),
                            preferred_element_type=jnp.float32)
        + b1_ref[...], 0.0)
    out_ref[...] = (
        jnp.dot(a.astype(jnp.bfloat16), w2_ref[...],
                preferred_element_type=jnp.float32)
        + b2_ref[...]).astype(jnp.bfloat16)


def _proj_body(x_ref, w_ref, b_ref, out_ref):
    out_ref[...] = jnp.maximum(
        jnp.dot(x_ref[...], w_ref[...], preferred_element_type=jnp.float32)
        + b_ref[...], 0.0)


def _msg_body(hs_ref, ew_ref, r_ref, m_ref):
    g = hs_ref.shape[0]
    hsw = hs_ref[...].astype(jnp.bfloat16)
    eww = ew_ref[...]
    outs = []
    for j in range(4):
        hj = hsw[:, j * _D:(j + 1) * _D]
        hrep = pltpu.repeat(hj, _D, axis=1)
        pj = hrep * eww[j * g:(j + 1) * g, :]
        outs.append(jnp.dot(pj, r_ref[...],
                            preferred_element_type=jnp.float32))
    m_ref[...] = jnp.concatenate(outs, axis=1)


def _combine_body(p_ref, b_ref, out_ref):
    out_ref[...] = jnp.maximum(p_ref[0] + p_ref[1] + b_ref[...], 0.0)


_MESH = plsc.VectorSubcoreMesh(core_axis_name="c", subcore_axis_name="s")
_SC_PARAMS = pltpu.CompilerParams(use_tc_tiling_on_sc=False)


@functools.partial(
    pl.kernel,
    mesh=_MESH,
    compiler_params=_SC_PARAMS,
    out_type=jax.ShapeDtypeStruct((_E_PAD, _D), jnp.float32),
    scratch_types=[
        pltpu.VMEM((_CHUNKS, _CH), jnp.int32),
        pltpu.VMEM((_WROWS, _D), jnp.float32),
        pltpu.VMEM((_WROWS, _D), jnp.float32),
        pltpu.VMEM_SHARED((_V_PAD, _D), jnp.float32),
        pltpu.SemaphoreType.DMA,
        pltpu.SemaphoreType.DMA,
    ],
)
def _gather(h_hbm, idx_hbm, hs_hbm, idx_v, buf0, buf1, h_sh, gsem, wsem):
    sid = lax.axis_index("s")
    wid = sid * _NC + lax.axis_index("c")
    stage = pltpu.async_copy(
        h_hbm.at[pl.ds(sid * _RPW, _RPW)], h_sh.at[pl.ds(sid * _RPW, _RPW)],
        wsem)
    pltpu.sync_copy(idx_hbm.at[wid], idx_v)
    stage.wait()
    plsc.subcore_barrier()
    base = wid * _EPW
    bufs = (buf0, buf1)
    wdesc = [None, None]
    for w in range(_WAVES):
        buf = bufs[w % 2]
        if wdesc[w % 2] is not None:
            wdesc[w % 2].wait()
        descs = []
        for j in range(_WCH):
            c = w * _WCH + j
            descs.append(pltpu.async_copy(
                h_sh.at[idx_v.at[c]], buf.at[pl.ds(j * _CH, _CH)], gsem))
        for dsc in descs:
            dsc.wait()
        wdesc[w % 2] = pltpu.async_copy(
            buf, hs_hbm.at[pl.ds(base + w * _WROWS, _WROWS)], wsem)
    for d in wdesc:
        if d is not None:
            d.wait()


@functools.partial(
    pl.kernel,
    mesh=_MESH,
    compiler_params=_SC_PARAMS,
    out_type=jax.ShapeDtypeStruct((_E_PAD, _D), jnp.float32),
    scratch_types=[
        pltpu.VMEM((_CHUNKS, _CH), jnp.int32),
        pltpu.VMEM((_WROWS, _D), jnp.float32),
        pltpu.VMEM((_WROWS, _D), jnp.float32),
        pltpu.VMEM((_RPW, _D), jnp.float32),
        pltpu.VMEM((2, 16), jnp.float32),
        pltpu.VMEM_SHARED((_V_PAD, _D), jnp.float32),
        pltpu.SemaphoreType.DMA,
        pltpu.SemaphoreType.DMA,
    ],
)
def _gather_fused(p_hbm, bias_hbm, idx_hbm, hs_hbm, idx_v, buf0, buf1,
                  pb0, bbuf, h_sh, gsem, wsem):
    sid = lax.axis_index("s")
    wid = sid * _NC + lax.axis_index("c")
    pltpu.sync_copy(p_hbm.at[0].at[pl.ds(sid * _RPW, _RPW)], pb0)
    pltpu.sync_copy(p_hbm.at[1].at[pl.ds(sid * _RPW, _RPW)],
                    buf0.at[pl.ds(0, _RPW)])
    pltpu.sync_copy(bias_hbm, bbuf)
    pltpu.sync_copy(idx_hbm.at[wid], idx_v)
    blo = bbuf[0, :]
    bhi = bbuf[1, :]

    def body(r, carry):
        v0 = jnp.maximum(pb0[r, pl.ds(0, 16)] + buf0[r, pl.ds(0, 16)] + blo,
                         0.0)
        pb0[r, pl.ds(0, 16)] = v0
        v1 = jnp.maximum(pb0[r, pl.ds(16, 16)] + buf0[r, pl.ds(16, 16)] + bhi,
                         0.0)
        pb0[r, pl.ds(16, 16)] = v1
        return carry

    lax.fori_loop(0, _RPW, body, 0)
    pltpu.sync_copy(pb0, h_sh.at[pl.ds(sid * _RPW, _RPW)])
    plsc.subcore_barrier()
    base = wid * _EPW
    bufs = (buf0, buf1)
    wdesc = [None, None]
    for w in range(_WAVES):
        buf = bufs[w % 2]
        if wdesc[w % 2] is not None:
            wdesc[w % 2].wait()
        descs = []
        for j in range(_WCH):
            c = w * _WCH + j
            descs.append(pltpu.async_copy(
                h_sh.at[idx_v.at[c]], buf.at[pl.ds(j * _CH, _CH)], gsem))
        for dsc in descs:
            dsc.wait()
        wdesc[w % 2] = pltpu.async_copy(
            buf, hs_hbm.at[pl.ds(base + w * _WROWS, _WROWS)], wsem)
    for d in wdesc:
        if d is not None:
            d.wait()


@functools.partial(
    pl.kernel,
    mesh=_MESH,
    compiler_params=_SC_PARAMS,
    out_type=jax.ShapeDtypeStruct((_NC, _V_PAD, _D), jnp.float32),
    scratch_types=[
        pltpu.VMEM((_CHUNKS, _CH), jnp.int32),
        pltpu.VMEM((_WROWS, _D), jnp.float32),
        pltpu.VMEM((_WROWS, _D), jnp.float32),
        pltpu.VMEM((_RPW, _D), jnp.float32),
        pltpu.VMEM_SHARED((_V_PAD, _D), jnp.float32),
        pltpu.SemaphoreType.DMA,
        pltpu.SemaphoreType.DMA,
    ],
)
def _scatter(m_hbm, idx_hbm, zeros_hbm, out_hbm, idx_v, buf0, buf1, tbuf_v,
             acc_sh, lsem, ssem):
    cid = lax.axis_index("c")
    sid = lax.axis_index("s")
    wid = sid * _NC + cid
    base = wid * _EPW
    bufs = (buf0, buf1)
    pltpu.sync_copy(idx_hbm.at[wid], idx_v)
    pltpu.sync_copy(zeros_hbm.at[pl.ds(sid * _RPW, _RPW)], tbuf_v)
    pltpu.sync_copy(tbuf_v, acc_sh.at[pl.ds(sid * _RPW, _RPW)])

    ld = [None, None]
    ld[0] = pltpu.async_copy(m_hbm.at[pl.ds(base, _WROWS)], buf0, lsem)
    plsc.subcore_barrier()
    adds = [[], []]
    for w in range(_WAVES):
        p = w % 2
        if w + 1 < _WAVES:
            pn = (w + 1) % 2
            for d in adds[pn]:
                d.wait()
            adds[pn] = []
            ld[pn] = pltpu.async_copy(
                m_hbm.at[pl.ds(base + (w + 1) * _WROWS, _WROWS)],
                bufs[pn], lsem)
        ld[p].wait()
        for j in range(_WCH):
            adds[p].append(pltpu.async_copy(
                bufs[p].at[pl.ds(j * _CH, _CH)],
                acc_sh.at[idx_v.at[w * _WCH + j]], ssem, add=True))
    for par in adds:
        for d in par:
            d.wait()
    plsc.subcore_barrier()
    pltpu.sync_copy(acc_sh.at[pl.ds(sid * _RPW, _RPW)], tbuf_v)
    pltpu.sync_copy(tbuf_v, out_hbm.at[cid].at[pl.ds(sid * _RPW, _RPW)])


def kernel(node_feats, edge_feats, W_proj, b_proj, W1, b1, W2, b2, bias,
           edge_index):
    f32 = jnp.float32
    src = edge_index[0]
    dst = edge_index[1]

    cols = jnp.arange(_D * _D, dtype=jnp.int32)
    perm = (cols % _D) * _D + cols // _D
    W2p = W2[:, perm]
    b2p = b2[perm]

    def _psi_perm(arr, fill):
        ext = jnp.concatenate(
            [arr, jnp.full((_E_PAD - _E,), fill, jnp.int32)])
        return ext.reshape(_E_PAD // _BM, 4, _G).transpose(0, 2, 1).reshape(
            _NW, _CHUNKS, _CH)


    src_rs = _psi_perm(src, 0)
    dst_rs = _psi_perm(dst, _V)

    r_mat = jnp.kron(jnp.eye(_D, dtype=jnp.bfloat16),
                     jnp.ones((_D, 1), dtype=jnp.bfloat16))
    zeros_acc = jnp.zeros((_V_PAD, _D), dtype=f32)

    be = 1280
    ew3 = pl.pallas_call(
        _ew_body,
        grid=(_E // be,),
        in_specs=[
            pl.BlockSpec((_EDGE_IN, be), lambda i: (0, i)),
            pl.BlockSpec((_EDGE_IN, _EH), lambda i: (0, 0)),
            pl.BlockSpec((1, _EH), lambda i: (0, 0)),
            pl.BlockSpec((_EH, _D * _D), lambda i: (0, 0)),
            pl.BlockSpec((1, _D * _D), lambda i: (0, 0)),
        ],
        out_specs=pl.BlockSpec((be, _D * _D), lambda i: (i, 0)),
        out_shape=jax.ShapeDtypeStruct((_E_PAD, _D * _D), jnp.bfloat16),
    )(edge_feats.T, W1, b1.reshape(1, _EH), W2p.astype(jnp.bfloat16),
      b2p.reshape(1, _D * _D))

    h = pl.pallas_call(
        _proj_body,
        grid=(1,),
        in_specs=[
            pl.BlockSpec((_V, _NODE_IN), lambda i: (0, 0)),
            pl.BlockSpec((_NODE_IN, _D), lambda i: (0, 0)),
            pl.BlockSpec((1, _D), lambda i: (0, 0)),
        ],
        out_specs=pl.BlockSpec((_V, _D), lambda i: (0, 0)),
        out_shape=jax.ShapeDtypeStruct((_V_PAD, _D), f32),
    )(node_feats, W_proj, b_proj.reshape(1, _D))

    msg = pl.pallas_call(
        _msg_body,
        grid=(_E_PAD // _BM,),
        in_specs=[
            pl.BlockSpec((_G, _D * 4), lambda i: (i, 0)),
            pl.BlockSpec((_BM, _D * _D), lambda i: (i, 0)),
            pl.BlockSpec((_D * _D, _D), lambda i: (0, 0)),
        ],
        out_specs=pl.BlockSpec((_G, _D * 4), lambda i: (i, 0)),
        out_shape=jax.ShapeDtypeStruct((_E_PAD // 4, _D * 4), f32),
    )

    vq = _V_PAD // 4
    combine = pl.pallas_call(
        _combine_body,
        grid=(1,),
        in_specs=[
            pl.BlockSpec((_NC, vq, 128), lambda i: (0, 0, 0)),
            pl.BlockSpec((1, 128), lambda i: (0, 0)),
        ],
        out_specs=pl.BlockSpec((vq, 128), lambda i: (0, 0)),
        out_shape=jax.ShapeDtypeStruct((vq, 128), f32),
    )

    bias_t = jnp.tile(bias, 4).reshape(1, 128)
    bias_2x16 = bias.reshape(2, 16)
    p = None
    for s in range(_STEPS):
        if s == 0:
            hs = _gather(h, src_rs)
        else:
            hs = _gather_fused(p, bias_2x16, src_rs)
        m_w = msg(hs.reshape(_E_PAD // 4, _D * 4), ew3, r_mat)
        p = _scatter(m_w.reshape(_E_PAD, _D), dst_rs, zeros_acc)
    h = combine(p.reshape(_NC, vq, 128), bias_t).reshape(_V_PAD, _D)
    return h[:_V]
